# Initial kernel scaffold; baseline (speedup 1.0000x reference)
import jax
import jax.numpy as jnp
from jax.experimental import pallas as pl


def kernel(x, edge_index, edge_weight, noise, W1, b1, W2, b2, W3, b3):
    raise NotImplementedError("write your pallas kernel here")


# speedup vs baseline: 3.7162x; 3.7162x over previous
import functools

import jax
import jax.numpy as jnp
from jax.experimental import pallas as pl
from jax.experimental.pallas import tpu as pltpu

N_NODES = 10000
N_EDGES = 320000
IN_DIM = 128
H1 = 256
H2 = 64

BN = 1000


def _t1_body(degp_ref, x_ref, dinv_ref, y1_ref):
    deg = degp_ref[:, 0:1] + degp_ref[:, 1:2] + 1.0
    dinv = jax.lax.rsqrt(deg)
    dinv_ref[...] = dinv
    y1_ref[...] = x_ref[...] * dinv


def _t1(degp, x):
    return pl.pallas_call(
        _t1_body,
        grid=(N_NODES // BN,),
        in_specs=[
            pl.BlockSpec((BN, 2), lambda i: (i, 0)),
            pl.BlockSpec((BN, IN_DIM), lambda i: (i, 0)),
        ],
        out_specs=[
            pl.BlockSpec((BN, 1), lambda i: (i, 0)),
            pl.BlockSpec((BN, IN_DIM), lambda i: (i, 0)),
        ],
        out_shape=[
            jax.ShapeDtypeStruct((N_NODES, 1), jnp.float32),
            jax.ShapeDtypeStruct((N_NODES, IN_DIM), jnp.float32),
        ],
    )(degp, x)


def _t2_body(acc_ref, y1_ref, dinv_ref, w1_ref, b1_ref, w23_ref, y2_ref):
    dinv = dinv_ref[...]
    ax = dinv * (acc_ref[...] + y1_ref[...])
    h = jax.nn.relu(
        jnp.dot(ax, w1_ref[...], preferred_element_type=jnp.float32)
        + b1_ref[...]
    )
    hc = jnp.dot(h, w23_ref[...], preferred_element_type=jnp.float32)
    y2_ref[...] = hc * dinv


def _t2(acc, y1, dinv, W1, b1, W23):
    return pl.pallas_call(
        _t2_body,
        grid=(N_NODES // BN,),
        in_specs=[
            pl.BlockSpec((BN, IN_DIM), lambda i: (i, 0)),
            pl.BlockSpec((BN, IN_DIM), lambda i: (i, 0)),
            pl.BlockSpec((BN, 1), lambda i: (i, 0)),
            pl.BlockSpec((IN_DIM, H1), lambda i: (0, 0)),
            pl.BlockSpec((1, H1), lambda i: (0, 0)),
            pl.BlockSpec((H1, 2 * H2), lambda i: (0, 0)),
        ],
        out_specs=pl.BlockSpec((BN, 2 * H2), lambda i: (i, 0)),
        out_shape=jax.ShapeDtypeStruct((N_NODES, 2 * H2), jnp.float32),
    )(acc, y1, dinv, W1, b1, W23)


def _t3_body(acc_ref, y2_ref, dinv_ref, b23_ref, noise_ref,
             z_ref, mean_ref, ls_ref):
    out2 = dinv_ref[...] * (acc_ref[...] + y2_ref[...]) + b23_ref[...]
    mean = out2[:, :H2]
    log_std = out2[:, H2:]
    mean_ref[...] = mean
    ls_ref[...] = log_std
    z_ref[...] = mean + noise_ref[...] * jnp.exp(log_std)


def _t3(acc, y2, dinv, b23, noise):
    return pl.pallas_call(
        _t3_body,
        grid=(N_NODES // BN,),
        in_specs=[
            pl.BlockSpec((BN, 2 * H2), lambda i: (i, 0)),
            pl.BlockSpec((BN, 2 * H2), lambda i: (i, 0)),
            pl.BlockSpec((BN, 1), lambda i: (i, 0)),
            pl.BlockSpec((1, 2 * H2), lambda i: (0, 0)),
            pl.BlockSpec((BN, H2), lambda i: (i, 0)),
        ],
        out_specs=[
            pl.BlockSpec((BN, H2), lambda i: (i, 0)),
            pl.BlockSpec((BN, H2), lambda i: (i, 0)),
            pl.BlockSpec((BN, H2), lambda i: (i, 0)),
        ],
        out_shape=[
            jax.ShapeDtypeStruct((N_NODES, H2), jnp.float32),
            jax.ShapeDtypeStruct((N_NODES, H2), jnp.float32),
            jax.ShapeDtypeStruct((N_NODES, H2), jnp.float32),
        ],
    )(acc, y2, dinv, b23, noise)


def kernel(x, edge_index, edge_weight, noise, W1, b1, W2, b2, W3, b3):
    src = edge_index[0].astype(jnp.int32)
    dst = edge_index[1].astype(jnp.int32)
    ew = edge_weight

    deg = jnp.zeros((N_NODES,), jnp.float32).at[dst].add(ew)
    degp = jnp.stack([deg, jnp.zeros_like(deg)], axis=1)

    dinv, y1 = _t1(degp, x)

    acc1 = jnp.zeros((N_NODES, IN_DIM), jnp.float32).at[dst].add(
        ew[:, None] * y1[src])

    W23 = jnp.concatenate([W2, W3], axis=1)
    y2 = _t2(acc1, y1, dinv, W1, b1.reshape(1, H1), W23)

    acc2 = jnp.zeros((N_NODES, 2 * H2), jnp.float32).at[dst].add(
        ew[:, None] * y2[src])

    b23 = jnp.concatenate([b2, b3]).reshape(1, 2 * H2)
    z, mean, log_std = _t3(acc2, y2, dinv, b23, noise)
    return (z, mean, log_std)


# speedup vs baseline: 33.6750x; 9.0617x over previous
import functools

import jax
import jax.numpy as jnp
from jax import lax
from jax.experimental import pallas as pl
from jax.experimental.pallas import tpu as pltpu
from jax.experimental.pallas import tpu_sc as plsc

N_NODES = 10000
N_EDGES = 320000
IN_DIM = 128
H1 = 256
H2 = 64

BN = 1000

NC = 2
NS = 16
NW = NC * NS
CK = 128
NCHUNK = 80
E_PAD = NW * NCHUNK * CK
N_PAD = 10240
RPS = N_PAD // NS

_SC_MESH = plsc.VectorSubcoreMesh(core_axis_name="c", subcore_axis_name="s")


@functools.partial(
    pl.kernel,
    out_type=[jax.ShapeDtypeStruct((N_PAD,), jnp.float32),
              jax.ShapeDtypeStruct((N_PAD,), jnp.float32)],
    mesh=_SC_MESH,
    scratch_types=[
        pltpu.VMEM((NCHUNK, CK), jnp.int32),
        pltpu.VMEM((NCHUNK, CK), jnp.float32),
        pltpu.VMEM_SHARED((N_PAD,), jnp.float32),
    ],
)
def _sc_deg(dst_hbm, ew_hbm, zeros_hbm, out0_hbm, out1_hbm,
            dst_v, ew_v, deg_sh):
    c = lax.axis_index("c")
    s = lax.axis_index("s")
    wid = s * NC + c
    pltpu.sync_copy(dst_hbm.at[wid], dst_v)
    pltpu.sync_copy(ew_hbm.at[wid], ew_v)

    @pl.when(s == 0)
    def _():
        pltpu.sync_copy(zeros_hbm, deg_sh)

    plsc.subcore_barrier()

    def body(k, carry):
        pltpu.sync_copy(ew_v.at[k], deg_sh.at[dst_v.at[k]], add=True)
        return carry

    lax.fori_loop(0, NCHUNK, body, 0)
    plsc.subcore_barrier()

    @pl.when(c == 0)
    def _():
        pltpu.sync_copy(deg_sh.at[pl.ds(s * RPS, RPS)],
                        out0_hbm.at[pl.ds(s * RPS, RPS)])

    @pl.when(c == 1)
    def _():
        pltpu.sync_copy(deg_sh.at[pl.ds(s * RPS, RPS)],
                        out1_hbm.at[pl.ds(s * RPS, RPS)])


@functools.partial(
    pl.kernel,
    out_type=[jax.ShapeDtypeStruct((N_PAD, IN_DIM), jnp.float32),
              jax.ShapeDtypeStruct((N_PAD, IN_DIM), jnp.float32)],
    mesh=_SC_MESH,
    scratch_types=[
        pltpu.VMEM((3, 2, CK), jnp.int32),
        pltpu.VMEM((3, CK), jnp.float32),
        pltpu.VMEM((2, CK, IN_DIM), jnp.float32),
        pltpu.VMEM_SHARED((N_PAD, IN_DIM), jnp.float32),
        pltpu.SemaphoreType.DMA,
        pltpu.SemaphoreType.DMA,
    ],
)
def _sc_pass(y_hbm, e_hbm, w_hbm, zrows_hbm, out0_hbm, out1_hbm,
             e_v, w_v, rows_v, acc_sh, esem, gsem):
    c = lax.axis_index("c")
    s = lax.axis_index("s")
    wid = s * NC + c
    ew_hbm = e_hbm.at[wid]
    wf_hbm = w_hbm.at[wid]
    pltpu.sync_copy(zrows_hbm.at[pl.ds(s * RPS, RPS)],
                    acc_sh.at[pl.ds(s * RPS, RPS)])
    plsc.subcore_barrier()

    pltpu.sync_copy(ew_hbm.at[0], e_v.at[0])
    pltpu.sync_copy(wf_hbm.at[0], w_v.at[0])
    pltpu.async_copy(y_hbm.at[e_v.at[0, 0]], rows_v.at[0], gsem)
    pltpu.async_copy(ew_hbm.at[1], e_v.at[1], esem)
    pltpu.async_copy(wf_hbm.at[1], w_v.at[1], esem)

    def step(k, eb, rb):
        @pl.when(k + 1 < NCHUNK)
        def _():
            nb = (k + 1) % 3
            pltpu.make_async_copy(ew_hbm.at[k + 1], e_v.at[nb], esem).wait()
            pltpu.make_async_copy(wf_hbm.at[k + 1], w_v.at[nb], esem).wait()
            pltpu.async_copy(y_hbm.at[e_v.at[nb, 0]], rows_v.at[1 - rb], gsem)

        @pl.when(k + 2 < NCHUNK)
        def _():
            nb2 = (k + 2) % 3
            pltpu.async_copy(ew_hbm.at[k + 2], e_v.at[nb2], esem)
            pltpu.async_copy(wf_hbm.at[k + 2], w_v.at[nb2], esem)

        pltpu.make_async_copy(y_hbm.at[e_v.at[eb, 0]], rows_v.at[rb],
                              gsem).wait()

        def scale(g, carry2):
            wv = w_v[eb, pl.ds(g * 16, 16)]
            for i in range(16):
                w = wv[i]
                row = g * 16 + i
                for f in range(IN_DIM // 16):
                    sl = pl.ds(f * 16, 16)
                    rows_v[rb, row, sl] = rows_v[rb, row, sl] * w
            return carry2

        lax.fori_loop(0, CK // 16, scale, 0)
        pltpu.sync_copy(rows_v.at[rb], acc_sh.at[e_v.at[eb, 1]], add=True)

    def body(j, carry):
        for b in (0, 1):
            k = 2 * j + b
            step(k, k % 3, b)
        return carry

    lax.fori_loop(0, NCHUNK // 2, body, 0)
    plsc.subcore_barrier()

    @pl.when(c == 0)
    def _():
        pltpu.sync_copy(acc_sh.at[pl.ds(s * RPS, RPS)],
                        out0_hbm.at[pl.ds(s * RPS, RPS)])

    @pl.when(c == 1)
    def _():
        pltpu.sync_copy(acc_sh.at[pl.ds(s * RPS, RPS)],
                        out1_hbm.at[pl.ds(s * RPS, RPS)])


def _t1_body(degp_ref, x_ref, dinv_ref, y1_ref):
    deg = degp_ref[:, 0:1] + degp_ref[:, 1:2] + 1.0
    dinv = jax.lax.rsqrt(deg)
    dinv_ref[...] = dinv
    y1_ref[...] = x_ref[...] * dinv


def _t1(degp, x):
    return pl.pallas_call(
        _t1_body,
        grid=(N_NODES // BN,),
        in_specs=[
            pl.BlockSpec((BN, 2), lambda i: (i, 0)),
            pl.BlockSpec((BN, IN_DIM), lambda i: (i, 0)),
        ],
        out_specs=[
            pl.BlockSpec((BN, 1), lambda i: (i, 0)),
            pl.BlockSpec((BN, IN_DIM), lambda i: (i, 0)),
        ],
        out_shape=[
            jax.ShapeDtypeStruct((N_NODES, 1), jnp.float32),
            jax.ShapeDtypeStruct((N_NODES, IN_DIM), jnp.float32),
        ],
    )(degp, x)


def _t2_body(acc_ref, y1_ref, dinv_ref, w1_ref, b1_ref, w23_ref, y2_ref):
    dinv = dinv_ref[...]
    ax = dinv * (acc_ref[...] + y1_ref[...])
    h = jax.nn.relu(
        jnp.dot(ax, w1_ref[...], preferred_element_type=jnp.float32)
        + b1_ref[...]
    )
    hc = jnp.dot(h, w23_ref[...], preferred_element_type=jnp.float32)
    y2_ref[...] = hc * dinv


def _t2(acc, y1, dinv, W1, b1, W23):
    return pl.pallas_call(
        _t2_body,
        grid=(N_NODES // BN,),
        in_specs=[
            pl.BlockSpec((BN, IN_DIM), lambda i: (i, 0)),
            pl.BlockSpec((BN, IN_DIM), lambda i: (i, 0)),
            pl.BlockSpec((BN, 1), lambda i: (i, 0)),
            pl.BlockSpec((IN_DIM, H1), lambda i: (0, 0)),
            pl.BlockSpec((1, H1), lambda i: (0, 0)),
            pl.BlockSpec((H1, 2 * H2), lambda i: (0, 0)),
        ],
        out_specs=pl.BlockSpec((BN, 2 * H2), lambda i: (i, 0)),
        out_shape=jax.ShapeDtypeStruct((N_NODES, 2 * H2), jnp.float32),
    )(acc, y1, dinv, W1, b1, W23)


def _t3_body(acc_ref, y2_ref, dinv_ref, b23_ref, noise_ref,
             z_ref, mean_ref, ls_ref):
    out2 = dinv_ref[...] * (acc_ref[...] + y2_ref[...]) + b23_ref[...]
    mean = out2[:, :H2]
    log_std = out2[:, H2:]
    mean_ref[...] = mean
    ls_ref[...] = log_std
    z_ref[...] = mean + noise_ref[...] * jnp.exp(log_std)


def _t3(acc, y2, dinv, b23, noise):
    return pl.pallas_call(
        _t3_body,
        grid=(N_NODES // BN,),
        in_specs=[
            pl.BlockSpec((BN, 2 * H2), lambda i: (i, 0)),
            pl.BlockSpec((BN, 2 * H2), lambda i: (i, 0)),
            pl.BlockSpec((BN, 1), lambda i: (i, 0)),
            pl.BlockSpec((1, 2 * H2), lambda i: (0, 0)),
            pl.BlockSpec((BN, H2), lambda i: (i, 0)),
        ],
        out_specs=[
            pl.BlockSpec((BN, H2), lambda i: (i, 0)),
            pl.BlockSpec((BN, H2), lambda i: (i, 0)),
            pl.BlockSpec((BN, H2), lambda i: (i, 0)),
        ],
        out_shape=[
            jax.ShapeDtypeStruct((N_NODES, H2), jnp.float32),
            jax.ShapeDtypeStruct((N_NODES, H2), jnp.float32),
            jax.ShapeDtypeStruct((N_NODES, H2), jnp.float32),
        ],
    )(acc, y2, dinv, b23, noise)


def kernel(x, edge_index, edge_weight, noise, W1, b1, W2, b2, W3, b3):
    n_pad = E_PAD - N_EDGES
    pad_idx = jnp.arange(n_pad, dtype=jnp.int32) % N_NODES
    src = jnp.concatenate([edge_index[0].astype(jnp.int32), pad_idx])
    dst = jnp.concatenate([edge_index[1].astype(jnp.int32), pad_idx])
    ew = jnp.concatenate([edge_weight, jnp.zeros((n_pad,), jnp.float32)])
    e3 = jnp.stack([src.reshape(NW, NCHUNK, CK),
                    dst.reshape(NW, NCHUNK, CK)], axis=2)
    w3 = ew.reshape(NW, NCHUNK, CK)
    dst3d = dst.reshape(NW, NCHUNK, CK)
    ew3d = ew.reshape(NW, NCHUNK, CK)
    zeros_n = jnp.zeros((N_PAD,), jnp.float32)
    zeros_nf = jnp.zeros((N_PAD, IN_DIM), jnp.float32)

    deg0, deg1 = _sc_deg(dst3d, ew3d, zeros_n)
    degp_t = jnp.stack([deg0[:N_NODES], deg1[:N_NODES]], axis=1)

    dinv, y1 = _t1(degp_t, x)

    a0, a1 = _sc_pass(y1, e3, w3, zeros_nf)
    acc1 = a0[:N_NODES] + a1[:N_NODES]

    W23 = jnp.concatenate([W2, W3], axis=1)
    y2 = _t2(acc1, y1, dinv, W1, b1.reshape(1, H1), W23)

    c0, c1 = _sc_pass(y2, e3, w3, zeros_nf)
    acc2 = c0[:N_NODES] + c1[:N_NODES]

    b23 = jnp.concatenate([b2, b3]).reshape(1, 2 * H2)
    z, mean, log_std = _t3(acc2, y2, dinv, b23, noise)
    return (z, mean, log_std)
